# baseline (speedup 1.0000x reference)
import functools

import jax
import jax.numpy as jnp
from jax.experimental import pallas as pl
from jax.experimental.pallas import tpu as pltpu

KV = 27
E_PER_K = 23000
C_IN = 128
C_OUT = 128
EPS = 1e-5

ROWS_PER_BLK = 1000


def _mm_body(g_ref, w_ref, o_ref):
    o_ref[0] = jnp.dot(g_ref[0], w_ref[0], preferred_element_type=jnp.float32)


def _batched_mm(gathered, weight):
    grid = (KV, E_PER_K // ROWS_PER_BLK)
    return pl.pallas_call(
        _mm_body,
        grid=grid,
        in_specs=[
            pl.BlockSpec((1, ROWS_PER_BLK, C_IN), lambda k, j: (k, j, 0)),
            pl.BlockSpec((1, C_IN, C_OUT), lambda k, j: (k, 0, 0)),
        ],
        out_specs=pl.BlockSpec((1, ROWS_PER_BLK, C_OUT), lambda k, j: (k, j, 0)),
        out_shape=jax.ShapeDtypeStruct((KV, E_PER_K, C_OUT), jnp.float32),
    )(gathered, weight)


def kernel(input_feat, input_coord, input_cmap, input_kmap, weight, bn_weight, bn_bias):
    N = input_feat.shape[0]
    src = input_kmap[0].reshape(KV, E_PER_K)
    dst = input_kmap[1].reshape(KV, E_PER_K)
    gathered = jnp.take(input_feat, src, axis=0)
    msgs = _batched_mm(gathered, weight)
    out = jnp.zeros((N, C_OUT), dtype=jnp.float32).at[dst.reshape(-1)].add(
        msgs.reshape(-1, C_OUT))
    mean = jnp.mean(out, axis=0)
    var = jnp.mean((out - mean) ** 2, axis=0)
    out = (out - mean) * jax.lax.rsqrt(var + EPS) * bn_weight + bn_bias
    return jax.nn.relu(out)


# speedup vs baseline: 1.8208x; 1.8208x over previous
import functools

import jax
import jax.numpy as jnp
from jax import lax
from jax.experimental import pallas as pl
from jax.experimental.pallas import tpu as pltpu
from jax.experimental.pallas import tpu_sc as plsc

KV = 27
E_PER_K = 23000
C = 128
EPS = 1e-5
N_VOX = 50000

NC = 2
NS = 16
CH = 128

EPK_PAD = 23040
E_FLAT = KV * EPK_PAD
E_PAD = 622592
BIG = 1 << 30

RANGE = 12544
N_OUT = 4 * RANGE
DUMP = RANGE
ACC_ROWS = RANGE + 16
ROWS_PER_TILE = RANGE // NS

MM_BLK = 512
MM_J = EPK_PAD // MM_BLK

_mesh = plsc.VectorSubcoreMesh(
    core_axis_name="c", subcore_axis_name="s", num_cores=NC, num_subcores=NS)


@functools.partial(
    pl.kernel,
    mesh=_mesh,
    out_type=jax.ShapeDtypeStruct((E_PAD, C), jnp.float32),
    scratch_types=[
        pltpu.VMEM((CH,), jnp.int32),
        pltpu.VMEM((CH, C), jnp.float32),
        pltpu.SemaphoreType.DMA,
    ],
)
def _sc_gather(feat_hbm, src_hbm, out_hbm, idx_v, rows_v, sem):
    wid = lax.axis_index("s") * NC + lax.axis_index("c")
    n_ch = E_PAD // (NC * NS * CH)
    base = wid * n_ch * CH

    def body(i, carry):
        off = base + i * CH
        pltpu.sync_copy(src_hbm.at[pl.ds(off, CH)], idx_v)
        pltpu.async_copy(feat_hbm.at[idx_v], rows_v, sem).wait()
        pltpu.sync_copy(rows_v, out_hbm.at[pl.ds(off, CH)])
        return carry

    lax.fori_loop(0, n_ch, body, 0)


def _mm_body(g_ref, w_ref, o_ref):
    o_ref[...] = jnp.dot(g_ref[...], w_ref[0],
                         preferred_element_type=jnp.float32)


def _batched_mm(gathered, weight):
    return pl.pallas_call(
        _mm_body,
        grid=(KV, MM_J),
        in_specs=[
            pl.BlockSpec((MM_BLK, C), lambda k, j: (k * MM_J + j, 0)),
            pl.BlockSpec((1, C, C), lambda k, j: (k, 0, 0)),
        ],
        out_specs=pl.BlockSpec((MM_BLK, C), lambda k, j: (k * MM_J + j, 0)),
        out_shape=jax.ShapeDtypeStruct((E_PAD, C), jnp.float32),
    )(gathered, weight)


@functools.partial(
    pl.kernel,
    mesh=_mesh,
    out_type=jax.ShapeDtypeStruct((N_OUT, C), jnp.float32),
    scratch_types=[
        pltpu.VMEM((CH,), jnp.int32),
        pltpu.VMEM((CH,), jnp.int32),
        pltpu.VMEM((CH, C), jnp.float32),
        pltpu.VMEM_SHARED((ACC_ROWS, C), jnp.float32),
        pltpu.SemaphoreType.DMA,
    ],
)
def _sc_scatter(msgs_hbm, dst_hbm, zeros_hbm, out_hbm,
                idx_v, idx2_v, rows_v, acc, sem):
    cid = lax.axis_index("c")
    sid = lax.axis_index("s")
    per_tile = E_PAD // NS
    n_ch = per_tile // CH
    ebase = sid * per_tile

    for p in range(2):
        rng = 2 * p + cid
        lo = rng * RANGE

        pltpu.sync_copy(zeros_hbm, acc.at[pl.ds(sid * ROWS_PER_TILE,
                                                ROWS_PER_TILE)])
        plsc.subcore_barrier()

        def body(i, carry):
            off = ebase + i * CH
            pltpu.sync_copy(dst_hbm.at[pl.ds(off, CH)], idx_v)
            pltpu.async_copy(msgs_hbm.at[pl.ds(off, CH)], rows_v, sem).wait()
            for j in range(CH // 16):
                d = idx_v[pl.ds(j * 16, 16)]
                ok = (d >= lo) & (d < lo + RANGE)
                idx2_v[pl.ds(j * 16, 16)] = jnp.where(ok, d - lo, DUMP)
            pltpu.sync_copy(rows_v, acc.at[idx2_v], add=True)
            return carry

        lax.fori_loop(0, n_ch, body, 0)
        plsc.subcore_barrier()

        pltpu.sync_copy(
            acc.at[pl.ds(sid * ROWS_PER_TILE, ROWS_PER_TILE)],
            out_hbm.at[pl.ds(lo + sid * ROWS_PER_TILE, ROWS_PER_TILE)])
        plsc.subcore_barrier()


def _stats_body(x_ref, s_ref, q_ref):
    pid = pl.program_id(0)
    row = lax.broadcasted_iota(jnp.int32, (MM_BLK, 1), 0) + pid * MM_BLK
    x = jnp.where(row < N_VOX, x_ref[...], 0.0)

    @pl.when(pid == 0)
    def _():
        s_ref[...] = jnp.zeros_like(s_ref)
        q_ref[...] = jnp.zeros_like(q_ref)

    s_ref[...] += jnp.sum(x, axis=0, keepdims=True)
    q_ref[...] += jnp.sum(x * x, axis=0, keepdims=True)


def _apply_body(x_ref, s_ref, q_ref, w_ref, b_ref, o_ref):
    inv_n = 1.0 / N_VOX
    mean = s_ref[...] * inv_n
    var = q_ref[...] * inv_n - mean * mean
    scale = lax.rsqrt(var + EPS) * w_ref[...]
    o_ref[...] = jnp.maximum((x_ref[...] - mean) * scale + b_ref[...], 0.0)


def _bn_relu(out_acc, bn_weight, bn_bias):
    nblk = N_OUT // MM_BLK
    s, q = pl.pallas_call(
        _stats_body,
        grid=(nblk,),
        in_specs=[pl.BlockSpec((MM_BLK, C), lambda i: (i, 0))],
        out_specs=[pl.BlockSpec((1, C), lambda i: (0, 0)),
                   pl.BlockSpec((1, C), lambda i: (0, 0))],
        out_shape=[jax.ShapeDtypeStruct((1, C), jnp.float32),
                   jax.ShapeDtypeStruct((1, C), jnp.float32)],
    )(out_acc)
    return pl.pallas_call(
        _apply_body,
        grid=(nblk,),
        in_specs=[
            pl.BlockSpec((MM_BLK, C), lambda i: (i, 0)),
            pl.BlockSpec((1, C), lambda i: (0, 0)),
            pl.BlockSpec((1, C), lambda i: (0, 0)),
            pl.BlockSpec((1, C), lambda i: (0, 0)),
            pl.BlockSpec((1, C), lambda i: (0, 0)),
        ],
        out_specs=pl.BlockSpec((MM_BLK, C), lambda i: (i, 0)),
        out_shape=jax.ShapeDtypeStruct((N_OUT, C), jnp.float32),
    )(out_acc, s, q, bn_weight.reshape(1, C), bn_bias.reshape(1, C))


def kernel(input_feat, input_coord, input_cmap, input_kmap, weight,
           bn_weight, bn_bias):
    src = input_kmap[0].reshape(KV, E_PER_K)
    dst = input_kmap[1].reshape(KV, E_PER_K)

    src_p = jnp.zeros((KV, EPK_PAD), jnp.int32).at[:, :E_PER_K].set(src)
    src_p = jnp.concatenate(
        [src_p.reshape(-1), jnp.zeros((E_PAD - E_FLAT,), jnp.int32)])
    dst_p = jnp.full((KV, EPK_PAD), BIG, jnp.int32).at[:, :E_PER_K].set(dst)
    dst_p = jnp.concatenate(
        [dst_p.reshape(-1), jnp.full((E_PAD - E_FLAT,), BIG, jnp.int32)])

    gathered = _sc_gather(input_feat, src_p)
    msgs = _batched_mm(gathered, weight)
    zeros = jnp.zeros((ROWS_PER_TILE, C), jnp.float32)
    out_acc = _sc_scatter(msgs, dst_p, zeros)
    y = _bn_relu(out_acc, bn_weight, bn_bias)
    return y[:N_VOX]


# speedup vs baseline: 2.1234x; 1.1662x over previous
import functools

import jax
import jax.numpy as jnp
from jax import lax
from jax.experimental import pallas as pl
from jax.experimental.pallas import tpu as pltpu
from jax.experimental.pallas import tpu_sc as plsc

KV = 27
E_PER_K = 23000
C = 128
EPS = 1e-5
N_VOX = 50000

NC = 2
NS = 16
CH = 128

EPK_PAD = 23040
E_FLAT = KV * EPK_PAD
E_PAD = 622592
BIG = 1 << 30

RANGE = 12544
N_OUT = 4 * RANGE
DUMP = RANGE
ACC_ROWS = RANGE + 16
ROWS_PER_TILE = RANGE // NS

MM_BLK = 512
MM_J = EPK_PAD // MM_BLK

_mesh = plsc.VectorSubcoreMesh(
    core_axis_name="c", subcore_axis_name="s", num_cores=NC, num_subcores=NS)


@functools.partial(
    pl.kernel,
    mesh=_mesh,
    out_type=jax.ShapeDtypeStruct((E_PAD, C), jnp.float32),
    scratch_types=[
        pltpu.VMEM((CH,), jnp.int32),
        pltpu.VMEM((CH, C), jnp.float32),
        pltpu.SemaphoreType.DMA,
    ],
)
def _sc_gather(feat_hbm, src_hbm, out_hbm, idx_v, rows_v, sem):
    wid = lax.axis_index("s") * NC + lax.axis_index("c")
    n_ch = E_PAD // (NC * NS * CH)
    base = wid * n_ch * CH

    def body(i, carry):
        off = base + i * CH
        pltpu.sync_copy(src_hbm.at[pl.ds(off, CH)], idx_v)
        pltpu.async_copy(feat_hbm.at[idx_v], rows_v, sem).wait()
        pltpu.sync_copy(rows_v, out_hbm.at[pl.ds(off, CH)])
        return carry

    lax.fori_loop(0, n_ch, body, 0)


def _mm_body(g_ref, w_ref, o_ref):
    o_ref[...] = jnp.dot(g_ref[...], w_ref[0],
                         preferred_element_type=jnp.float32)


def _batched_mm(gathered, weight):
    return pl.pallas_call(
        _mm_body,
        grid=(KV, MM_J),
        in_specs=[
            pl.BlockSpec((MM_BLK, C), lambda k, j: (k * MM_J + j, 0)),
            pl.BlockSpec((1, C, C), lambda k, j: (k, 0, 0)),
        ],
        out_specs=pl.BlockSpec((MM_BLK, C), lambda k, j: (k * MM_J + j, 0)),
        out_shape=jax.ShapeDtypeStruct((E_PAD, C), jnp.float32),
    )(gathered, weight)


BUF = 416


@functools.partial(
    pl.kernel,
    mesh=_mesh,
    out_type=jax.ShapeDtypeStruct((N_OUT, C), jnp.float32),
    scratch_types=[
        pltpu.VMEM((CH,), jnp.int32),
        pltpu.VMEM((BUF,), jnp.int32),
        pltpu.VMEM((BUF,), jnp.int32),
        pltpu.VMEM((1, CH), jnp.int32),
        pltpu.VMEM((1, CH), jnp.int32),
        pltpu.VMEM((CH, C), jnp.float32),
        pltpu.VMEM((16,), jnp.int32),
        pltpu.VMEM_SHARED((ACC_ROWS, C), jnp.float32),
        pltpu.SemaphoreType.DMA,
    ],
    compiler_params=pltpu.CompilerParams(needs_layout_passes=False),
)
def _sc_scatter(msgs_hbm, dst_hbm, zeros_hbm, out_hbm,
                idx_v, eidx_b, ldst_b, fire_e, fire_d, grows, spill, acc,
                sem):
    cid = lax.axis_index("c")
    sid = lax.axis_index("s")
    per_tile = E_PAD // NS
    n_ch = per_tile // CH
    ebase = sid * per_tile

    def fire_and_shift():
        for t in range(CH // 16):
            fire_e[0, pl.ds(16 * t, 16)] = eidx_b[pl.ds(16 * t, 16)]
            fire_d[0, pl.ds(16 * t, 16)] = ldst_b[pl.ds(16 * t, 16)]
        pltpu.async_copy(msgs_hbm.at[fire_e.at[0]], grows, sem).wait()
        pltpu.sync_copy(grows, acc.at[fire_d.at[0]], add=True)
        for t in range(CH // 16):
            eidx_b[pl.ds(16 * t, 16)] = eidx_b[pl.ds(CH + 16 * t, 16)]
            ldst_b[pl.ds(16 * t, 16)] = ldst_b[pl.ds(CH + 16 * t, 16)]

    def pad_dump(cnt):
        zeros16 = jnp.zeros((16,), jnp.int32)
        dump16 = jnp.full((16,), DUMP, jnp.int32)
        for t in range(CH // 16):
            eidx_b[pl.ds(cnt + 16 * t, 16)] = zeros16
            ldst_b[pl.ds(cnt + 16 * t, 16)] = dump16

    lane = lax.iota(jnp.int32, 16)

    for p in range(2):
        rng = 2 * p + cid
        lo = rng * RANGE

        pltpu.sync_copy(zeros_hbm, acc.at[pl.ds(sid * ROWS_PER_TILE,
                                                ROWS_PER_TILE)])
        plsc.subcore_barrier()

        def body(i, cnt):
            off = ebase + i * CH
            pltpu.sync_copy(dst_hbm.at[pl.ds(off, CH)], idx_v)
            cnt_v = jnp.broadcast_to(cnt, (16,))
            for j in range(CH // 16):
                d = idx_v[pl.ds(j * 16, 16)]
                ok = (d >= lo) & (d < lo + RANGE)
                scan = plsc.cumsum(ok.astype(jnp.int32))
                pos = cnt_v + scan - 1
                plsc.store_scatter(eidx_b, [pos], off + 16 * j + lane,
                                   mask=ok)
                plsc.store_scatter(ldst_b, [pos], d - lo, mask=ok)
                cnt_v = cnt_v + plsc.all_reduce_population_count(ok)
            cnt = cnt_v[0]

            @pl.when(cnt >= CH)
            def _():
                fire_and_shift()

            return jnp.where(cnt >= CH, cnt - CH, cnt)

        cnt = lax.fori_loop(0, n_ch, body, jnp.int32(0))
        pad_dump(cnt)
        fire_and_shift()
        pad_dump(jnp.maximum(cnt - CH, 0))
        fire_and_shift()
        plsc.subcore_barrier()

        pltpu.sync_copy(
            acc.at[pl.ds(sid * ROWS_PER_TILE, ROWS_PER_TILE)],
            out_hbm.at[pl.ds(lo + sid * ROWS_PER_TILE, ROWS_PER_TILE)])
        plsc.subcore_barrier()


def _stats_body(x_ref, s_ref, q_ref):
    pid = pl.program_id(0)
    row = lax.broadcasted_iota(jnp.int32, (MM_BLK, 1), 0) + pid * MM_BLK
    x = jnp.where(row < N_VOX, x_ref[...], 0.0)

    @pl.when(pid == 0)
    def _():
        s_ref[...] = jnp.zeros_like(s_ref)
        q_ref[...] = jnp.zeros_like(q_ref)

    s_ref[...] += jnp.sum(x, axis=0, keepdims=True)
    q_ref[...] += jnp.sum(x * x, axis=0, keepdims=True)


def _apply_body(x_ref, s_ref, q_ref, w_ref, b_ref, o_ref):
    inv_n = 1.0 / N_VOX
    mean = s_ref[...] * inv_n
    var = q_ref[...] * inv_n - mean * mean
    scale = lax.rsqrt(var + EPS) * w_ref[...]
    o_ref[...] = jnp.maximum((x_ref[...] - mean) * scale + b_ref[...], 0.0)


def _bn_relu(out_acc, bn_weight, bn_bias):
    nblk = N_OUT // MM_BLK
    s, q = pl.pallas_call(
        _stats_body,
        grid=(nblk,),
        in_specs=[pl.BlockSpec((MM_BLK, C), lambda i: (i, 0))],
        out_specs=[pl.BlockSpec((1, C), lambda i: (0, 0)),
                   pl.BlockSpec((1, C), lambda i: (0, 0))],
        out_shape=[jax.ShapeDtypeStruct((1, C), jnp.float32),
                   jax.ShapeDtypeStruct((1, C), jnp.float32)],
    )(out_acc)
    return pl.pallas_call(
        _apply_body,
        grid=(nblk,),
        in_specs=[
            pl.BlockSpec((MM_BLK, C), lambda i: (i, 0)),
            pl.BlockSpec((1, C), lambda i: (0, 0)),
            pl.BlockSpec((1, C), lambda i: (0, 0)),
            pl.BlockSpec((1, C), lambda i: (0, 0)),
            pl.BlockSpec((1, C), lambda i: (0, 0)),
        ],
        out_specs=pl.BlockSpec((MM_BLK, C), lambda i: (i, 0)),
        out_shape=jax.ShapeDtypeStruct((N_OUT, C), jnp.float32),
    )(out_acc, s, q, bn_weight.reshape(1, C), bn_bias.reshape(1, C))


def kernel(input_feat, input_coord, input_cmap, input_kmap, weight,
           bn_weight, bn_bias):
    src = input_kmap[0].reshape(KV, E_PER_K)
    dst = input_kmap[1].reshape(KV, E_PER_K)

    src_p = jnp.zeros((KV, EPK_PAD), jnp.int32).at[:, :E_PER_K].set(src)
    src_p = jnp.concatenate(
        [src_p.reshape(-1), jnp.zeros((E_PAD - E_FLAT,), jnp.int32)])
    dst_p = jnp.full((KV, EPK_PAD), BIG, jnp.int32).at[:, :E_PER_K].set(dst)
    dst_p = jnp.concatenate(
        [dst_p.reshape(-1), jnp.full((E_PAD - E_FLAT,), BIG, jnp.int32)])

    gathered = _sc_gather(input_feat, src_p)
    msgs = _batched_mm(gathered, weight)
    zeros = jnp.zeros((ROWS_PER_TILE, C), jnp.float32)
    out_acc = _sc_scatter(msgs, dst_p, zeros)
    y = _bn_relu(out_acc, bn_weight, bn_bias)
    return y[:N_VOX]
